# Initial kernel scaffold; baseline (speedup 1.0000x reference)
import jax
import jax.numpy as jnp
from jax.experimental import pallas as pl


def kernel(x):
    raise NotImplementedError("write your pallas kernel here")


# speedup vs baseline: 1.1322x; 1.1322x over previous
import functools

import jax
import jax.numpy as jnp
from jax.experimental import pallas as pl
from jax.experimental.pallas import tpu as pltpu

_B, _N, _D = 16, 4096, 512
_CHUNK = 1024
_NC = _N // _CHUNK


def _reduce_kernel(x_ref, out_ref, cnt_ref):
    c = pl.program_id(1)

    @pl.when(c == 0)
    def _init():
        out_ref[...] = jnp.zeros_like(out_ref)
        cnt_ref[0, 0] = 0.0

    blk = x_ref[0]
    out_ref[0] += jnp.sum(blk, axis=0, keepdims=True)
    valid = jnp.any(blk != 0, axis=-1)
    cnt_ref[0, 0] += jnp.sum(valid.astype(jnp.float32))

    @pl.when(c == _NC - 1)
    def _finish():
        out_ref[...] = out_ref[...] / cnt_ref[0, 0]


@jax.jit
def kernel(x):
    out = pl.pallas_call(
        _reduce_kernel,
        grid=(_B, _NC),
        in_specs=[pl.BlockSpec((1, _CHUNK, _D), lambda b, c: (b, c, 0))],
        out_specs=pl.BlockSpec((1, 1, _D), lambda b, c: (b, 0, 0)),
        out_shape=jax.ShapeDtypeStruct((_B, 1, _D), jnp.float32),
        scratch_shapes=[pltpu.SMEM((1, 1), jnp.float32)],
    )(x)
    return out[:, 0, :]
